# baseline (device time: 116320 ns/iter reference)
import jax
import jax.numpy as jnp
from jax import lax
from jax.experimental import pallas as pl
from jax.experimental.pallas import tpu as pltpu

N_DEV = 4
W_CHUNKS = 8


def kernel(x, w_mat, scale_x, scale_w):
    m_per, k = x.shape
    _, n_total = w_mat.shape
    n_per = n_total // N_DEV
    m_total = m_per * N_DEV
    half = m_per // 2
    kc = k // W_CHUNKS

    def body(x_ref, w_ref, sx_ref, sw_ref, out_ref,
             xg, w8, xstage, wstage, stage,
             xs_sems, ws_sems, rs_sems, rr_sems, ls_sems, lr_sems,
             copy_sems):
        my = lax.axis_index("i")
        left = lax.rem(my + (N_DEV - 1), N_DEV)
        right = lax.rem(my + 1, N_DEV)

        barrier = pltpu.get_barrier_semaphore()
        for nbr in (left, right):
            pl.semaphore_signal(barrier, inc=1, device_id=(nbr,),
                                device_id_type=pl.DeviceIdType.MESH)
        pl.semaphore_wait(barrier, 2)

        scale = sx_ref[0] * sw_ref[0]

        def a_rows(o):
            return pl.ds(o * m_per, half)

        def b_rows(o):
            return pl.ds(o * m_per + half, half)

        xcps = []
        for c in range(2):
            cp = pltpu.make_async_copy(
                x_ref.at[pl.ds(c * half, half), :],
                xstage.at[c], xs_sems.at[c])
            cp.start()
            xcps.append(cp)

        wcps = [None] * W_CHUNKS

        def start_wchunk(c):
            cp = pltpu.make_async_copy(
                w_ref.at[pl.ds(c * kc, kc), pl.ds(my * n_per, n_per)],
                wstage.at[c % 2], ws_sems.at[c % 2])
            cp.start()
            wcps[c] = cp

        start_wchunk(0)
        start_wchunk(1)

        def make_hop(h):
            o_r = lax.rem(my + (N_DEV - h), N_DEV)
            o_l = lax.rem(my + h, N_DEV)
            rdma_r = pltpu.make_async_remote_copy(
                src_ref=xg.at[a_rows(o_r), :], dst_ref=xg.at[a_rows(o_r), :],
                send_sem=rs_sems.at[h], recv_sem=rr_sems.at[h],
                device_id=(right,), device_id_type=pl.DeviceIdType.MESH)
            rdma_l = pltpu.make_async_remote_copy(
                src_ref=xg.at[b_rows(o_l), :], dst_ref=xg.at[b_rows(o_l), :],
                send_sem=ls_sems.at[h], recv_sem=lr_sems.at[h],
                device_id=(left,), device_id_type=pl.DeviceIdType.MESH)
            return rdma_r, rdma_l

        hop = make_hop(0)
        xcps[0].wait()
        xg[a_rows(my), :] = xstage[0].astype(jnp.float8_e4m3fn)
        hop[0].start()
        xcps[1].wait()
        xg[b_rows(my), :] = xstage[1].astype(jnp.float8_e4m3fn)
        hop[1].start()

        for c in range(W_CHUNKS):
            wcps[c].wait()
            w8[pl.ds(c * kc, kc), :] = wstage[c % 2].astype(jnp.float8_e4m3fn)
            if c + 2 < W_CHUNKS:
                start_wchunk(c + 2)

        pending = [None, None]
        slot = [0]

        def compute_half(rows, out_row):
            s = slot[0] & 1
            slot[0] += 1
            if pending[s] is not None:
                pending[s].wait()
            acc = lax.dot_general(
                xg[rows, :], w8[...],
                dimension_numbers=(((1,), (0,)), ((), ())),
                preferred_element_type=jnp.float32,
            )
            stage[s] = jnp.maximum(acc * scale, 0.0)
            cp = pltpu.make_async_copy(
                stage.at[s], out_ref.at[pl.ds(out_row, half), :],
                copy_sems.at[s])
            cp.start()
            pending[s] = cp

        o_am1 = lax.rem(my + (N_DEV - 1), N_DEV)
        o_ap1 = lax.rem(my + 1, N_DEV)
        o_2 = lax.rem(my + 2, N_DEV)

        windows = {
            1: [(a_rows(my), my * m_per),
                (b_rows(my), my * m_per + half),
                (a_rows(o_am1), o_am1 * m_per)],
            2: [(b_rows(o_ap1), o_ap1 * m_per + half),
                (a_rows(o_2), o_2 * m_per),
                (b_rows(o_2), o_2 * m_per + half)],
        }

        for h in range(N_DEV - 1):
            for rows, out_row in windows.get(h, []):
                compute_half(rows, out_row)
            rdma_r, rdma_l = hop
            rdma_r.wait_recv()
            rdma_l.wait_recv()
            rdma_r.wait_send()
            rdma_l.wait_send()
            if h + 1 < N_DEV - 1:
                hop = make_hop(h + 1)
                hop[0].start()
                hop[1].start()

        compute_half(a_rows(o_ap1), o_ap1 * m_per)
        compute_half(b_rows(o_am1), o_am1 * m_per + half)

        for p in pending:
            if p is not None:
                p.wait()

    return pl.pallas_call(
        body,
        out_shape=jax.ShapeDtypeStruct((m_total, n_per), jnp.float32),
        in_specs=[
            pl.BlockSpec(memory_space=pl.ANY),
            pl.BlockSpec(memory_space=pl.ANY),
            pl.BlockSpec(memory_space=pltpu.SMEM),
            pl.BlockSpec(memory_space=pltpu.SMEM),
        ],
        out_specs=pl.BlockSpec(memory_space=pl.ANY),
        scratch_shapes=[
            pltpu.VMEM((m_total, k), jnp.float8_e4m3fn),
            pltpu.VMEM((k, n_per), jnp.float8_e4m3fn),
            pltpu.VMEM((2, half, k), jnp.float32),
            pltpu.VMEM((2, kc, n_per), jnp.float32),
            pltpu.VMEM((2, half, n_per), jnp.float32),
            pltpu.SemaphoreType.DMA((2,)),
            pltpu.SemaphoreType.DMA((2,)),
            pltpu.SemaphoreType.DMA((N_DEV - 1,)),
            pltpu.SemaphoreType.DMA((N_DEV - 1,)),
            pltpu.SemaphoreType.DMA((N_DEV - 1,)),
            pltpu.SemaphoreType.DMA((N_DEV - 1,)),
            pltpu.SemaphoreType.DMA((2,)),
        ],
        compiler_params=pltpu.CompilerParams(
            collective_id=0, vmem_limit_bytes=100 * 1024 * 1024),
    )(x, w_mat, scale_x, scale_w)
